# Initial kernel scaffold; baseline (speedup 1.0000x reference)
import jax
import jax.numpy as jnp
from jax.experimental import pallas as pl


def kernel(node_embeddings, edge_embeddings, edge_index, W1, b1, W2, b2):
    raise NotImplementedError("write your pallas kernel here")


# speedup vs baseline: 4.5277x; 4.5277x over previous
import functools
import math

import jax
import jax.numpy as jnp
from jax import lax
from jax.experimental import pallas as pl
from jax.experimental.pallas import tpu as pltpu
from jax.experimental.pallas import tpu_sc as plsc

N = 10000
E = 320000
H = 128

NC = 2
NS = 16
NW = NC * NS
EPW = E // NW
CHUNK = 80
NCHUNK = EPW // CHUNK
NPAD = 10240
ROWS_PER_TILE = NPAD // NS


def _sc_aggregate(node_embeddings, edge_embeddings, dst_r, src_r):
    mesh = plsc.VectorSubcoreMesh(core_axis_name="c", subcore_axis_name="s")

    @functools.partial(
        pl.kernel,
        out_type=jax.ShapeDtypeStruct((NC, NPAD, H), jnp.float32),
        mesh=mesh,
        scratch_types=[
            pltpu.VMEM((NCHUNK, CHUNK), jnp.int32),
            pltpu.VMEM((NCHUNK, CHUNK), jnp.int32),
            pltpu.VMEM((CHUNK, H), jnp.float32),
            pltpu.VMEM_SHARED((NPAD, H), jnp.float32),
            pltpu.SemaphoreType.DMA,
        ],
    )
    def k(node_hbm, edge_hbm, dst_hbm, src_hbm, out_hbm,
          dst_v, src_v, buf, acc, sem):
        c = lax.axis_index("c")
        s = lax.axis_index("s")
        wid = c * NS + s
        base = wid * EPW

        @pl.loop(0, CHUNK)
        def _(r):
            for cc in range(H // 16):
                buf[r, pl.ds(cc * 16, 16)] = jnp.zeros((16,), jnp.float32)

        for kk in range(ROWS_PER_TILE // CHUNK):
            pltpu.sync_copy(
                buf, acc.at[pl.ds(s * ROWS_PER_TILE + kk * CHUNK, CHUNK)]
            )
        plsc.subcore_barrier()

        pltpu.sync_copy(dst_hbm.at[wid], dst_v)
        pltpu.sync_copy(src_hbm.at[wid], src_v)

        @pl.loop(0, NCHUNK)
        def _(j):
            pltpu.sync_copy(edge_hbm.at[pl.ds(base + j * CHUNK, CHUNK)], buf)
            pltpu.async_copy(node_hbm.at[dst_v.at[j]], buf, sem, add=True).wait()
            @pl.loop(0, CHUNK)
            def _(r):
                for cc in range(H // 16):
                    v = buf[r, pl.ds(cc * 16, 16)]
                    buf[r, pl.ds(cc * 16, 16)] = jnp.maximum(v, 0.0)
            pltpu.sync_copy(buf, acc.at[src_v.at[j]], add=True)

        plsc.subcore_barrier()
        pltpu.sync_copy(
            acc.at[pl.ds(s * ROWS_PER_TILE, ROWS_PER_TILE)],
            out_hbm.at[c, pl.ds(s * ROWS_PER_TILE, ROWS_PER_TILE)],
        )

    return k(node_embeddings, edge_embeddings, dst_r, src_r)


def _mlp_body(node_ref, acc_ref, w1_ref, b1_ref, w2_ref, b2_ref, out_ref):
    h = node_ref[...] + acc_ref[0] + acc_ref[1]
    t = jnp.dot(h, w1_ref[...], preferred_element_type=jnp.float32) + b1_ref[...]
    g = t * 0.5 * (1.0 + lax.erf(t * (1.0 / math.sqrt(2.0))))
    out_ref[...] = (
        jnp.dot(g, w2_ref[...], preferred_element_type=jnp.float32) + b2_ref[...]
    )


def _tc_mlp(node_embeddings, acc, W1, b1, W2, b2):
    BR = 1000
    grid = (N // BR,)
    return pl.pallas_call(
        _mlp_body,
        grid=grid,
        in_specs=[
            pl.BlockSpec((BR, H), lambda i: (i, 0)),
            pl.BlockSpec((NC, BR, H), lambda i: (0, i, 0)),
            pl.BlockSpec((H, H // 2), lambda i: (0, 0)),
            pl.BlockSpec((1, H // 2), lambda i: (0, 0)),
            pl.BlockSpec((H // 2, H), lambda i: (0, 0)),
            pl.BlockSpec((1, H), lambda i: (0, 0)),
        ],
        out_specs=pl.BlockSpec((BR, H), lambda i: (i, 0)),
        out_shape=jax.ShapeDtypeStruct((N, H), jnp.float32),
    )(node_embeddings, acc, W1, b1.reshape(1, -1), W2, b2.reshape(1, -1))


@jax.jit
def kernel(node_embeddings, edge_embeddings, edge_index, W1, b1, W2, b2):
    idx32 = edge_index.astype(jnp.int32)
    src_r = idx32[0].reshape(NW, NCHUNK, CHUNK)
    dst_r = idx32[1].reshape(NW, NCHUNK, CHUNK)
    acc = _sc_aggregate(node_embeddings, edge_embeddings, dst_r, src_r)
    return _tc_mlp(node_embeddings, acc, W1, b1, W2, b2)


# speedup vs baseline: 6.9439x; 1.5336x over previous
import functools
import math

import jax
import jax.numpy as jnp
from jax import lax
from jax.experimental import pallas as pl
from jax.experimental.pallas import tpu as pltpu
from jax.experimental.pallas import tpu_sc as plsc

N = 10000
E = 320000
H = 128

NC = 2
NS = 16
NW = NC * NS
EPW = E // NW
CHUNK = 40
NCHUNK = EPW // CHUNK
NBUF = 5
NPAD = 10240
ROWS_PER_TILE = NPAD // NS


def _sc_aggregate(node_embeddings, edge_embeddings, pidx):
    mesh = plsc.VectorSubcoreMesh(core_axis_name="c", subcore_axis_name="s")

    @functools.partial(
        pl.kernel,
        out_type=jax.ShapeDtypeStruct((NC, NPAD, H), jnp.float32),
        mesh=mesh,
        scratch_types=[
            pltpu.VMEM((NBUF, 2, CHUNK), jnp.int32),
            pltpu.VMEM((NBUF, CHUNK, H), jnp.float32),
            pltpu.VMEM_SHARED((NPAD, H), jnp.float32),
            pltpu.SemaphoreType.DMA((NBUF,)),
            pltpu.SemaphoreType.DMA((NBUF,)),
            pltpu.SemaphoreType.DMA((NBUF,)),
            pltpu.SemaphoreType.DMA((NBUF,)),
        ],
    )
    def k(node_hbm, edge_hbm, pidx_hbm, out_hbm,
          ibuf, buf, acc, isem, esem, gsem, ssem):
        c = lax.axis_index("c")
        s = lax.axis_index("s")
        wid = c * NS + s
        base = wid * EPW

        @plsc.parallel_loop(0, CHUNK, unroll=2)
        def _(r):
            for cc in range(H // 16):
                buf[0, r, pl.ds(cc * 16, 16)] = jnp.zeros((16,), jnp.float32)

        for kk in range(ROWS_PER_TILE // CHUNK):
            pltpu.sync_copy(
                buf.at[0], acc.at[pl.ds(s * ROWS_PER_TILE + kk * CHUNK, CHUNK)]
            )
        plsc.subcore_barrier()

        def issue_load(j, b):
            pltpu.async_copy(pidx_hbm.at[wid, j], ibuf.at[b], isem.at[b])
            pltpu.async_copy(
                edge_hbm.at[pl.ds(base + j * CHUNK, CHUNK)], buf.at[b],
                esem.at[b],
            )

        def wait_load(j, b):
            pltpu.make_async_copy(
                pidx_hbm.at[wid, j], ibuf.at[b], isem.at[b]
            ).wait()
            pltpu.make_async_copy(
                edge_hbm.at[pl.ds(base + j * CHUNK, CHUNK)], buf.at[b],
                esem.at[b],
            ).wait()

        def issue_gather(b):
            pltpu.async_copy(
                node_hbm.at[ibuf.at[b, 0]], buf.at[b], gsem.at[b], add=True
            )

        def wait_gather(b):
            pltpu.make_async_copy(
                node_hbm.at[ibuf.at[b, 0]], buf.at[b], gsem.at[b]
            ).wait()

        def issue_scatter(b):
            pltpu.async_copy(
                buf.at[b], acc.at[ibuf.at[b, 1]], ssem.at[b], add=True
            )

        def wait_scatter(b):
            pltpu.make_async_copy(
                buf.at[b], acc.at[ibuf.at[b, 1]], ssem.at[b]
            ).wait()

        def relu(b):
            @plsc.parallel_loop(0, CHUNK, unroll=2)
            def _(r):
                for cc in range(H // 16):
                    v = buf[b, r, pl.ds(cc * 16, 16)]
                    buf[b, r, pl.ds(cc * 16, 16)] = jnp.maximum(v, 0.0)

        def body(j, b, e_ok: bool, g_ok: bool, s_ok: bool):
            b1, b2 = (b + 1) % NBUF, (b + 2) % NBUF
            if e_ok:
                if s_ok:
                    wait_scatter(b2)
                issue_load(j + 2, b2)
            if g_ok:
                wait_load(j + 1, b1)
                issue_gather(b1)
            wait_gather(b)
            relu(b)
            issue_scatter(b)

        issue_load(0, 0)
        issue_load(1, 1)
        wait_load(0, 0)
        issue_gather(0)
        for j in range(NBUF):
            body(j, j % NBUF, e_ok=(j + 2 < NCHUNK), g_ok=(j + 1 < NCHUNK),
                 s_ok=(j >= 3))

        @pl.loop(1, NCHUNK // NBUF - 1)
        def _(t):
            for b in range(NBUF):
                body(t * NBUF + b, b, e_ok=True, g_ok=True, s_ok=True)

        for j in range(NCHUNK - NBUF, NCHUNK):
            body(j, j % NBUF, e_ok=(j + 2 < NCHUNK), g_ok=(j + 1 < NCHUNK),
                 s_ok=(j >= 3))
        for j in range(NCHUNK - NBUF, NCHUNK):
            wait_scatter(j % NBUF)

        plsc.subcore_barrier()
        pltpu.sync_copy(
            acc.at[pl.ds(s * ROWS_PER_TILE, ROWS_PER_TILE)],
            out_hbm.at[c, pl.ds(s * ROWS_PER_TILE, ROWS_PER_TILE)],
        )

    return k(node_embeddings, edge_embeddings, pidx)


def _mlp_body(node_ref, acc_ref, w1_ref, b1_ref, w2_ref, b2_ref, out_ref):
    h = node_ref[...] + acc_ref[0] + acc_ref[1]
    t = jnp.dot(h, w1_ref[...], preferred_element_type=jnp.float32) + b1_ref[...]
    g = t * 0.5 * (1.0 + lax.erf(t * (1.0 / math.sqrt(2.0))))
    out_ref[...] = (
        jnp.dot(g, w2_ref[...], preferred_element_type=jnp.float32) + b2_ref[...]
    )


def _tc_mlp(node_embeddings, acc, W1, b1, W2, b2):
    BR = 1000
    grid = (N // BR,)
    return pl.pallas_call(
        _mlp_body,
        grid=grid,
        in_specs=[
            pl.BlockSpec((BR, H), lambda i: (i, 0)),
            pl.BlockSpec((NC, BR, H), lambda i: (0, i, 0)),
            pl.BlockSpec((H, H // 2), lambda i: (0, 0)),
            pl.BlockSpec((1, H // 2), lambda i: (0, 0)),
            pl.BlockSpec((H // 2, H), lambda i: (0, 0)),
            pl.BlockSpec((1, H), lambda i: (0, 0)),
        ],
        out_specs=pl.BlockSpec((BR, H), lambda i: (i, 0)),
        out_shape=jax.ShapeDtypeStruct((N, H), jnp.float32),
    )(node_embeddings, acc, W1, b1.reshape(1, -1), W2, b2.reshape(1, -1))


@jax.jit
def kernel(node_embeddings, edge_embeddings, edge_index, W1, b1, W2, b2):
    idx32 = edge_index.astype(jnp.int32)
    dst_r = idx32[1].reshape(NW, NCHUNK, CHUNK)
    src_r = idx32[0].reshape(NW, NCHUNK, CHUNK)
    pidx = jnp.stack([dst_r, src_r], axis=2)
    acc = _sc_aggregate(node_embeddings, edge_embeddings, pidx)
    return _tc_mlp(node_embeddings, acc, W1, b1, W2, b2)


# speedup vs baseline: 8.2171x; 1.1834x over previous
import functools
import math

import jax
import jax.numpy as jnp
from jax import lax
from jax.experimental import pallas as pl
from jax.experimental.pallas import tpu as pltpu
from jax.experimental.pallas import tpu_sc as plsc

N = 10000
E = 320000
H = 128

NC = 2
NS = 16
NW = NC * NS
EPW = E // NW
CHUNK = 40
NCHUNK = EPW // CHUNK
NBUF = 5
NPAD = 10240
ROWS_PER_TILE = NPAD // NS


def _sc_aggregate(node_embeddings, edge_embeddings, pidx):
    mesh = plsc.VectorSubcoreMesh(core_axis_name="c", subcore_axis_name="s")

    @functools.partial(
        pl.kernel,
        out_type=jax.ShapeDtypeStruct((NC, NPAD, H), jnp.float32),
        mesh=mesh,
        scratch_types=[
            pltpu.VMEM((NBUF, 2, CHUNK), jnp.int32),
            pltpu.VMEM((NBUF, CHUNK, H), jnp.float32),
            pltpu.VMEM_SHARED((NPAD, H), jnp.float32),
            pltpu.SemaphoreType.DMA((NBUF,)),
            pltpu.SemaphoreType.DMA((NBUF,)),
            pltpu.SemaphoreType.DMA((NBUF,)),
            pltpu.SemaphoreType.DMA((NBUF,)),
        ],
    )
    def k(node_hbm, edge_hbm, pidx_hbm, out_hbm,
          ibuf, buf, acc, isem, esem, gsem, ssem):
        c = lax.axis_index("c")
        s = lax.axis_index("s")
        wid = c * NS + s
        base = wid * EPW

        @plsc.parallel_loop(0, CHUNK, unroll=2)
        def _(r):
            for cc in range(H // 16):
                buf[0, r, pl.ds(cc * 16, 16)] = jnp.zeros((16,), jnp.float32)

        for kk in range(ROWS_PER_TILE // CHUNK):
            pltpu.sync_copy(
                buf.at[0], acc.at[pl.ds(s * ROWS_PER_TILE + kk * CHUNK, CHUNK)]
            )
        plsc.subcore_barrier()

        def issue_load(j, b):
            pltpu.async_copy(pidx_hbm.at[1, wid, j], ibuf.at[b, 0], isem.at[b])
            pltpu.async_copy(pidx_hbm.at[0, wid, j], ibuf.at[b, 1], isem.at[b])
            pltpu.async_copy(
                edge_hbm.at[pl.ds(base + j * CHUNK, CHUNK)], buf.at[b],
                esem.at[b],
            )

        def wait_load(j, b):
            pltpu.make_async_copy(
                pidx_hbm.at[1, wid, j], ibuf.at[b, 0], isem.at[b]
            ).wait()
            pltpu.make_async_copy(
                pidx_hbm.at[0, wid, j], ibuf.at[b, 1], isem.at[b]
            ).wait()
            pltpu.make_async_copy(
                edge_hbm.at[pl.ds(base + j * CHUNK, CHUNK)], buf.at[b],
                esem.at[b],
            ).wait()

        def issue_gather(b):
            pltpu.async_copy(
                node_hbm.at[ibuf.at[b, 0]], buf.at[b], gsem.at[b], add=True
            )

        def wait_gather(b):
            pltpu.make_async_copy(
                node_hbm.at[ibuf.at[b, 0]], buf.at[b], gsem.at[b]
            ).wait()

        def issue_scatter(b):
            pltpu.async_copy(
                buf.at[b], acc.at[ibuf.at[b, 1]], ssem.at[b], add=True
            )

        def wait_scatter(b):
            pltpu.make_async_copy(
                buf.at[b], acc.at[ibuf.at[b, 1]], ssem.at[b]
            ).wait()

        def relu(b):
            @plsc.parallel_loop(0, CHUNK, unroll=2)
            def _(r):
                for cc in range(H // 16):
                    v = buf[b, r, pl.ds(cc * 16, 16)]
                    buf[b, r, pl.ds(cc * 16, 16)] = jnp.maximum(v, 0.0)

        def body(j, b, e_ok: bool, g_ok: bool, s_ok: bool):
            b2, b3 = (b + 2) % NBUF, (b + 3) % NBUF
            if e_ok:
                if s_ok:
                    wait_scatter(b3)
                issue_load(j + 3, b3)
            if g_ok:
                wait_load(j + 2, b2)
                issue_gather(b2)
            wait_gather(b)
            relu(b)
            issue_scatter(b)

        issue_load(0, 0)
        issue_load(1, 1)
        issue_load(2, 2)
        wait_load(0, 0)
        issue_gather(0)
        wait_load(1, 1)
        issue_gather(1)
        for j in range(NBUF):
            body(j, j % NBUF, e_ok=(j + 3 < NCHUNK), g_ok=(j + 2 < NCHUNK),
                 s_ok=(j >= 2))

        @pl.loop(1, NCHUNK // NBUF - 1)
        def _(t):
            for b in range(NBUF):
                body(t * NBUF + b, b, e_ok=True, g_ok=True, s_ok=True)

        for j in range(NCHUNK - NBUF, NCHUNK):
            body(j, j % NBUF, e_ok=(j + 3 < NCHUNK), g_ok=(j + 2 < NCHUNK),
                 s_ok=(j >= 2))
        for j in range(NCHUNK - 2, NCHUNK):
            wait_scatter(j % NBUF)

        plsc.subcore_barrier()
        pltpu.sync_copy(
            acc.at[pl.ds(s * ROWS_PER_TILE, ROWS_PER_TILE)],
            out_hbm.at[c, pl.ds(s * ROWS_PER_TILE, ROWS_PER_TILE)],
        )

    return k(node_embeddings, edge_embeddings, pidx)


def _mlp_body(node_ref, acc_ref, w1_ref, b1_ref, w2_ref, b2_ref, out_ref):
    h = node_ref[...] + acc_ref[0] + acc_ref[1]
    t = jnp.dot(h, w1_ref[...], preferred_element_type=jnp.float32) + b1_ref[...]
    g = t * 0.5 * (1.0 + lax.erf(t * (1.0 / math.sqrt(2.0))))
    out_ref[...] = (
        jnp.dot(g, w2_ref[...], preferred_element_type=jnp.float32) + b2_ref[...]
    )


def _tc_mlp(node_embeddings, acc, W1, b1, W2, b2):
    BR = 1000
    grid = (N // BR,)
    return pl.pallas_call(
        _mlp_body,
        grid=grid,
        in_specs=[
            pl.BlockSpec((BR, H), lambda i: (i, 0)),
            pl.BlockSpec((NC, BR, H), lambda i: (0, i, 0)),
            pl.BlockSpec((H, H // 2), lambda i: (0, 0)),
            pl.BlockSpec((1, H // 2), lambda i: (0, 0)),
            pl.BlockSpec((H // 2, H), lambda i: (0, 0)),
            pl.BlockSpec((1, H), lambda i: (0, 0)),
        ],
        out_specs=pl.BlockSpec((BR, H), lambda i: (i, 0)),
        out_shape=jax.ShapeDtypeStruct((N, H), jnp.float32),
    )(node_embeddings, acc, W1, b1.reshape(1, -1), W2, b2.reshape(1, -1))


@jax.jit
def kernel(node_embeddings, edge_embeddings, edge_index, W1, b1, W2, b2):
    idx32 = edge_index.astype(jnp.int32)
    pidx = idx32.reshape(2, NW, NCHUNK, CHUNK)
    acc = _sc_aggregate(node_embeddings, edge_embeddings, pidx)
    return _tc_mlp(node_embeddings, acc, W1, b1, W2, b2)
